# baseline (speedup 1.0000x reference)
import functools

import jax
import jax.numpy as jnp
from jax import lax
from jax.experimental import pallas as pl
from jax.experimental.pallas import tpu as pltpu
from jax.experimental.pallas import tpu_sc as plsc

EMB = 64
NN = 10000
NE = 320000
EBLK = 2000
NEB = NE // EBLK
NBLK = 1000
NNB = NN // NBLK

NWORK = 32
GC = NE // NWORK
GCH = 1000
SC_F = 8
SEG_E = NE // 2
SEG_CH = 256
SEG_GRP = SEG_CH // 16
ACC = NN * SC_F


@functools.cache
def _mesh():
    return plsc.VectorSubcoreMesh(core_axis_name="c", subcore_axis_name="s")


def _embed_e_body(x_ref, w_ref, b_ref, o_ref):
    e = jnp.dot(x_ref[...], w_ref[...],
                preferred_element_type=jnp.float32) + b_ref[...]
    o_ref[...] = jnp.concatenate([e, jnp.zeros_like(e)], axis=1)


def _embed_e(edges, w, b):
    return pl.pallas_call(
        _embed_e_body,
        grid=(NEB,),
        in_specs=[
            pl.BlockSpec((EBLK, 16), lambda i: (i, 0)),
            pl.BlockSpec((16, EMB), lambda i: (0, 0)),
            pl.BlockSpec((1, EMB), lambda i: (0, 0)),
        ],
        out_specs=pl.BlockSpec((EBLK, 2 * EMB), lambda i: (i, 0)),
        out_shape=jax.ShapeDtypeStruct((NE, 2 * EMB), jnp.float32),
    )(edges, w, b)


def _embed_n_body(x_ref, w_ref, b_ref, w1_ref, w2_ref, n_ref, ab_ref):
    n = jnp.dot(x_ref[...], w_ref[...],
                preferred_element_type=jnp.float32) + b_ref[...]
    n_ref[...] = n
    ab_ref[...] = jnp.concatenate(
        [jnp.dot(n, w1_ref[...], preferred_element_type=jnp.float32),
         jnp.dot(n, w2_ref[...], preferred_element_type=jnp.float32)], axis=1)


def _embed_n(nodes, w, b, ew1, ew2):
    return pl.pallas_call(
        _embed_n_body,
        grid=(NNB,),
        in_specs=[
            pl.BlockSpec((NBLK, 128), lambda i: (i, 0)),
            pl.BlockSpec((128, EMB), lambda i: (0, 0)),
            pl.BlockSpec((1, EMB), lambda i: (0, 0)),
            pl.BlockSpec((EMB, EMB), lambda i: (0, 0)),
            pl.BlockSpec((EMB, EMB), lambda i: (0, 0)),
        ],
        out_specs=[pl.BlockSpec((NBLK, EMB), lambda i: (i, 0)),
                   pl.BlockSpec((NBLK, 2 * EMB), lambda i: (i, 0))],
        out_shape=[jax.ShapeDtypeStruct((NN, EMB), jnp.float32),
                   jax.ShapeDtypeStruct((NN, 2 * EMB), jnp.float32)],
    )(nodes, w, b, ew1, ew2)


def _glob0_body(g_ref, w_ref, b_ref, ew3_ref, eb_ref, nw3_ref, nb_ref,
                go_ref, ce_ref, cn_ref):
    g = jnp.dot(g_ref[...], w_ref[...],
                preferred_element_type=jnp.float32) + b_ref[...]
    go_ref[...] = g
    ce_ref[...] = jnp.dot(g, ew3_ref[...],
                          preferred_element_type=jnp.float32) + eb_ref[...]
    cn_ref[...] = jnp.dot(g, nw3_ref[...],
                          preferred_element_type=jnp.float32) + nb_ref[...]


def _glob0(globals_, w, b, ew3, eb, nw3, nb):
    full = lambda shp: pl.BlockSpec(shp, lambda: tuple(0 for _ in shp))
    return pl.pallas_call(
        _glob0_body,
        in_specs=[full((1, 16)), full((16, EMB)), full((1, EMB)),
                  full((EMB, EMB)), full((1, EMB)),
                  full((EMB, EMB)), full((1, EMB))],
        out_specs=[full((1, EMB))] * 3,
        out_shape=[jax.ShapeDtypeStruct((1, EMB), jnp.float32)] * 3,
    )(globals_, w, b, ew3, eb, nw3, nb)


def _glob_body(np_ref, ep_ref, g_ref, gw_ref, gb_ref,
               ew3_ref, eb_ref, nw3_ref, nb_ref, go_ref, ce_ref, cn_ref):
    agg_n = jnp.sum(np_ref[...], axis=(0, 1)).reshape(1, EMB)
    agg_e = jnp.sum(ep_ref[...], axis=(0, 1)).reshape(1, EMB)
    g = g_ref[...]
    cat = jnp.concatenate([agg_n, agg_e, g], axis=-1)
    g = g + jnp.maximum(
        jnp.dot(cat, gw_ref[...], preferred_element_type=jnp.float32)
        + gb_ref[...], 0.0)
    go_ref[...] = g
    ce_ref[...] = jnp.dot(g, ew3_ref[...],
                          preferred_element_type=jnp.float32) + eb_ref[...]
    cn_ref[...] = jnp.dot(g, nw3_ref[...],
                          preferred_element_type=jnp.float32) + nb_ref[...]


def _glob(npart, epart, g, gw, gb, ew3, eb, nw3, nb):
    full = lambda shp: pl.BlockSpec(shp, lambda: tuple(0 for _ in shp))
    return pl.pallas_call(
        _glob_body,
        in_specs=[full((NNB, 1, EMB)), full((NEB, 1, EMB)), full((1, EMB)),
                  full((3 * EMB, EMB)), full((1, EMB)),
                  full((EMB, EMB)), full((1, EMB)),
                  full((EMB, EMB)), full((1, EMB))],
        out_specs=[full((1, EMB))] * 3,
        out_shape=[jax.ShapeDtypeStruct((1, EMB), jnp.float32)] * 3,
    )(npart, epart, g, gw, gb, ew3, eb, nw3, nb)


def _edge_body(e_ref, gs_ref, gr_ref, w0_ref, ce_ref, eo_ref, ep_ref):
    e = e_ref[...][:, :EMB]
    z = (jnp.dot(e, w0_ref[...], preferred_element_type=jnp.float32)
         + gs_ref[...][:, :EMB] + gr_ref[...][:, EMB:] + ce_ref[...])
    e2 = e + jnp.maximum(z, 0.0)
    eo_ref[...] = jnp.concatenate([e2, jnp.zeros_like(e2)], axis=1)
    ep_ref[...] = jnp.sum(e2, axis=0).reshape(1, 1, EMB)


def _edge(e128, gs, gr, w0, ce):
    return pl.pallas_call(
        _edge_body,
        grid=(NEB,),
        in_specs=[
            pl.BlockSpec((EBLK, 2 * EMB), lambda i: (i, 0)),
            pl.BlockSpec((EBLK, 2 * EMB), lambda i: (i, 0)),
            pl.BlockSpec((EBLK, 2 * EMB), lambda i: (i, 0)),
            pl.BlockSpec((EMB, EMB), lambda i: (0, 0)),
            pl.BlockSpec((1, EMB), lambda i: (0, 0)),
        ],
        out_specs=[pl.BlockSpec((EBLK, 2 * EMB), lambda i: (i, 0)),
                   pl.BlockSpec((1, 1, EMB), lambda i: (i, 0, 0))],
        out_shape=[jax.ShapeDtypeStruct((NE, 2 * EMB), jnp.float32),
                   jax.ShapeDtypeStruct((NEB, 1, EMB), jnp.float32)],
    )(e128, gs, gr, w0, ce)


def _edge_dec_body(e_ref, gs_ref, gr_ref, w0_ref, ce_ref, dw_ref, db_ref,
                   do_ref):
    e = e_ref[...][:, :EMB]
    z = (jnp.dot(e, w0_ref[...], preferred_element_type=jnp.float32)
         + gs_ref[...][:, :EMB] + gr_ref[...][:, EMB:] + ce_ref[...])
    e2 = e + jnp.maximum(z, 0.0)
    do_ref[...] = jnp.dot(e2, dw_ref[...],
                          preferred_element_type=jnp.float32) + db_ref[...]


def _edge_dec(e128, gs, gr, w0, ce, dw, db):
    return pl.pallas_call(
        _edge_dec_body,
        grid=(NEB,),
        in_specs=[
            pl.BlockSpec((EBLK, 2 * EMB), lambda i: (i, 0)),
            pl.BlockSpec((EBLK, 2 * EMB), lambda i: (i, 0)),
            pl.BlockSpec((EBLK, 2 * EMB), lambda i: (i, 0)),
            pl.BlockSpec((EMB, EMB), lambda i: (0, 0)),
            pl.BlockSpec((1, EMB), lambda i: (0, 0)),
            pl.BlockSpec((EMB, 8), lambda i: (0, 0)),
            pl.BlockSpec((1, 8), lambda i: (0, 0)),
        ],
        out_specs=pl.BlockSpec((EBLK, 8), lambda i: (i, 0)),
        out_shape=jax.ShapeDtypeStruct((NE, 8), jnp.float32),
    )(e128, gs, gr, w0, ce, dw, db)


def _node_body(n_ref, sp_ref, rp_ref, v0_ref, v1_ref, v2_ref, cn_ref,
               ew1_ref, ew2_ref, no_ref, ab_ref, np_ref):
    sp = sp_ref[...]
    rp = rp_ref[...]
    sv = jnp.maximum(sp[0], sp[1])
    rv = jnp.maximum(rp[0], rp[1])
    sent = jnp.where(jnp.isfinite(sv), sv, 0.0)
    recv = jnp.where(jnp.isfinite(rv), rv, 0.0)
    n = n_ref[...]
    z = (jnp.dot(n, v0_ref[...], preferred_element_type=jnp.float32)
         + jnp.dot(sent, v1_ref[...], preferred_element_type=jnp.float32)
         + jnp.dot(recv, v2_ref[...], preferred_element_type=jnp.float32)
         + cn_ref[...])
    n2 = n + jnp.maximum(z, 0.0)
    no_ref[...] = n2
    ab_ref[...] = jnp.concatenate(
        [jnp.dot(n2, ew1_ref[...], preferred_element_type=jnp.float32),
         jnp.dot(n2, ew2_ref[...], preferred_element_type=jnp.float32)],
        axis=1)
    np_ref[...] = jnp.sum(n2, axis=0).reshape(1, 1, EMB)


def _node(n, sparts, rparts, v0, v1, v2, cn, ew1, ew2):
    return pl.pallas_call(
        _node_body,
        grid=(NNB,),
        in_specs=[
            pl.BlockSpec((NBLK, EMB), lambda i: (i, 0)),
            pl.BlockSpec((2, NBLK, EMB), lambda i: (0, i, 0)),
            pl.BlockSpec((2, NBLK, EMB), lambda i: (0, i, 0)),
            pl.BlockSpec((EMB, EMB), lambda i: (0, 0)),
            pl.BlockSpec((EMB, EMB), lambda i: (0, 0)),
            pl.BlockSpec((EMB, EMB), lambda i: (0, 0)),
            pl.BlockSpec((1, EMB), lambda i: (0, 0)),
            pl.BlockSpec((EMB, EMB), lambda i: (0, 0)),
            pl.BlockSpec((EMB, EMB), lambda i: (0, 0)),
        ],
        out_specs=[pl.BlockSpec((NBLK, EMB), lambda i: (i, 0)),
                   pl.BlockSpec((NBLK, 2 * EMB), lambda i: (i, 0)),
                   pl.BlockSpec((1, 1, EMB), lambda i: (i, 0, 0))],
        out_shape=[jax.ShapeDtypeStruct((NN, EMB), jnp.float32),
                   jax.ShapeDtypeStruct((NN, 2 * EMB), jnp.float32),
                   jax.ShapeDtypeStruct((NNB, 1, EMB), jnp.float32)],
    )(n, sparts, rparts, v0, v1, v2, cn, ew1, ew2)


def _sc_gather(ab, snd, rcv):
    @functools.partial(
        pl.kernel,
        out_type=[jax.ShapeDtypeStruct((NE, 2 * EMB), jnp.float32)] * 2,
        mesh=_mesh(),
        scratch_types=[
            pltpu.VMEM((GCH,), jnp.int32),
            pltpu.VMEM((GCH, 2 * EMB), jnp.float32),
            pltpu.SemaphoreType.DMA,
        ],
    )
    def k(ab_hbm, snd_hbm, rcv_hbm, gs_out, gr_out, idx_v, rows_v, sem):
        wid = lax.axis_index("s") * 2 + lax.axis_index("c")
        base = wid * GC

        def body(i, carry):
            off = base + i * GCH
            pltpu.sync_copy(snd_hbm.at[pl.ds(off, GCH)], idx_v)
            pltpu.async_copy(ab_hbm.at[idx_v], rows_v, sem).wait()
            pltpu.sync_copy(rows_v, gs_out.at[pl.ds(off, GCH)])
            pltpu.sync_copy(rcv_hbm.at[pl.ds(off, GCH)], idx_v)
            pltpu.async_copy(ab_hbm.at[idx_v], rows_v, sem).wait()
            pltpu.sync_copy(rows_v, gr_out.at[pl.ds(off, GCH)])
            return carry

        lax.fori_loop(0, GC // GCH, body, 0)

    return k(ab, snd, rcv)


def _sc_segmax(e128, sr):
    @functools.partial(
        pl.kernel,
        out_type=jax.ShapeDtypeStruct((2, 2, SC_F, ACC), jnp.float32),
        mesh=_mesh(),
        compiler_params=pltpu.CompilerParams(needs_layout_passes=False),
        scratch_types=[
            pltpu.VMEM((SEG_CH,), jnp.int32),
            pltpu.VMEM((SEG_CH * 2 * EMB,), jnp.float32),
            pltpu.VMEM((ACC,), jnp.float32),
        ],
    )
    def k(e_hbm, sr_hbm, out, ids_v, vals_v, acc_v):
        wid = lax.axis_index("s") * 2 + lax.axis_index("c")
        t = wid // 16
        fg = (wid // 2) % 8
        h = wid % 2
        lanes = lax.iota(jnp.int32, 16)
        neg = jnp.full((16,), -jnp.inf, jnp.float32)

        def init_body(i, carry):
            acc_v[pl.ds(i * 16, 16)] = neg
            return carry

        lax.fori_loop(0, ACC // 16, init_body, 0)

        def chunk_body(ci, carry):
            off = h * SEG_E + ci * SEG_CH
            pltpu.sync_copy(sr_hbm.at[t, pl.ds(off, SEG_CH)], ids_v)
            pltpu.sync_copy(
                e_hbm.at[pl.ds(off * 2 * EMB, SEG_CH * 2 * EMB)], vals_v)

            def group_body(gi, gcarry):
                ids8 = ids_v[pl.ds(gi * 16, 16)] * SC_F
                vbase = (lanes + gi * 16) * (2 * EMB) + fg * SC_F
                for f in range(SC_F):
                    col = plsc.load_gather(vals_v, [vbase + f])
                    aidx = ids8 + f
                    cur = plsc.load_gather(acc_v, [aidx])
                    new = jnp.maximum(col, cur)
                    plsc.store_scatter(acc_v, [aidx], new)
                    chk = plsc.load_gather(acc_v, [aidx])
                    ok = chk >= new

                    def wcond(okc):
                        return jnp.logical_not(jnp.all(okc))

                    def wbody(okc):
                        plsc.store_scatter(acc_v, [aidx], new,
                                           mask=jnp.logical_not(okc))
                        chk2 = plsc.load_gather(acc_v, [aidx])
                        return chk2 >= new

                    lax.while_loop(wcond, wbody, ok)
                return gcarry

            lax.fori_loop(0, SEG_GRP, group_body, 0)
            return carry

        lax.fori_loop(0, SEG_E // SEG_CH, chunk_body, 0)
        pltpu.sync_copy(acc_v, out.at[t, h, fg])

    return k(e128, sr)


def _parts(slab):
    return slab.reshape(2, SC_F, NN, SC_F).transpose(0, 2, 1, 3).reshape(
        2, NN, EMB)


def kernel(nodes, edges, globals_, senders, receivers, emb_node_W,
           emb_node_b, emb_edge_W, emb_edge_b, emb_glob_W, emb_glob_b,
           edge_W, edge_b, node_W, node_b, glob_W, glob_b, dec_W, dec_b):
    L = edge_W.shape[0]
    r1 = lambda v: v.reshape(1, -1)
    dwp = jnp.pad(dec_W, ((0, 0), (0, 7)))
    dbp = jnp.pad(dec_b.reshape(1, 1), ((0, 0), (0, 7)))

    sr = jnp.stack([senders, receivers])
    e128 = _embed_e(edges, emb_edge_W, r1(emb_edge_b))
    n, ab = _embed_n(nodes, emb_node_W, r1(emb_node_b),
                     edge_W[0, 64:128], edge_W[0, 128:192])
    g, ce, cn = _glob0(globals_, emb_glob_W, r1(emb_glob_b),
                       edge_W[0, 192:256], r1(edge_b[0]),
                       node_W[0, 192:256], r1(node_b[0]))

    for l in range(L):
        gs, gr = _sc_gather(ab, senders, receivers)
        if l < L - 1:
            e128, epart = _edge(e128, gs, gr, edge_W[l, 0:64], ce)
            slab = _sc_segmax(e128.reshape(-1), sr)
            n, ab, npart = _node(
                n, _parts(slab[0]), _parts(slab[1]),
                node_W[l, 0:64], node_W[l, 64:128], node_W[l, 128:192], cn,
                edge_W[l + 1, 64:128], edge_W[l + 1, 128:192])
            g, ce, cn = _glob(npart, epart, g, glob_W[l], r1(glob_b[l]),
                              edge_W[l + 1, 192:256], r1(edge_b[l + 1]),
                              node_W[l + 1, 192:256], r1(node_b[l + 1]))
        else:
            dec = _edge_dec(e128, gs, gr, edge_W[l, 0:64], ce, dwp, dbp)

    return dec[:, :1]
